# baseline (speedup 1.0000x reference)
import jax
import jax.numpy as jnp
from jax.experimental import pallas as pl

_N = 81
_NP = 96
_V = 10


def _adj_kernel(ei_ref, eit_ref, a_ref):
    f32 = jnp.float32
    e = ei_ref.shape[1]
    np_ = a_ref.shape[0]
    dst_row = ei_ref[1:2, :]
    src_col = eit_ref[:, 0:1]
    dst_col = eit_ref[:, 1:2]
    oh_dt = (jax.lax.broadcasted_iota(jnp.int32, (np_, e), 0)
             == dst_row).astype(f32)
    lane = jax.lax.broadcasted_iota(jnp.int32, (e, np_), 1)
    oh_s = (src_col == lane).astype(f32)
    oh_d = (dst_col == lane).astype(f32)
    a_u = jnp.dot(oh_dt, oh_s, preferred_element_type=f32, precision=jax.lax.Precision.DEFAULT)
    ir = jax.lax.broadcasted_iota(jnp.int32, (np_, np_), 0)
    ic = jax.lax.broadcasted_iota(jnp.int32, (np_, np_), 1)
    eye = ((ir == ic) & (ir < _N)).astype(f32)
    a_u = a_u + eye
    deg_row = jnp.dot(jnp.ones((1, e), f32), oh_d,
                      preferred_element_type=f32, precision=jax.lax.Precision.DEFAULT) + (ic[0:1, :] < _N).astype(f32)
    deg_col = jnp.dot(a_u, jnp.ones((np_, 1), f32),
                      preferred_element_type=f32, precision=jax.lax.Precision.DEFAULT)
    r_row = jax.lax.rsqrt(jnp.maximum(deg_row, 1.0))
    r_col = jax.lax.rsqrt(jnp.maximum(deg_col, 1.0))
    a_ref[...] = a_u * r_col * r_row


def _gcn_kernel(x_ref, a_ref, w_in_ref, b_in_ref, ws_ref, bs_ref,
                w_out_ref, b_out_ref, out_ref):
    f32 = jnp.float32
    sn = x_ref.shape[0]
    np_ = a_ref.shape[0]
    s = sn // np_
    xb = x_ref[...]
    oh = (xb == jax.lax.broadcasted_iota(jnp.int32, (sn, _V), 1)).astype(f32)
    h = jnp.maximum(
        jnp.dot(oh, w_in_ref[...], preferred_element_type=f32, precision=jax.lax.Precision.DEFAULT) + b_in_ref[...],
        0.0)
    a = a_ref[...]
    for l in range(ws_ref.shape[0]):
        w_l = ws_ref[l]
        b_l = bs_ref[l]
        parts = []
        for b in range(s):
            hw_b = jnp.dot(h[b * np_:(b + 1) * np_, :], w_l,
                           preferred_element_type=f32)
            agg_b = jnp.dot(a, hw_b, preferred_element_type=f32)
            parts.append(jnp.maximum(agg_b + b_l, 0.0))
        h = jnp.concatenate(parts, axis=0)
    out_ref[...] = (jnp.dot(h, w_out_ref[...], preferred_element_type=f32)
                    + b_out_ref[...])


def kernel(x, W_in, b_in, Ws, bs, W_out, b_out, edge_index):
    B = x.shape[0]
    H = W_in.shape[1]
    L = Ws.shape[0]
    n, np_ = _N, _NP

    a = pl.pallas_call(
        _adj_kernel,
        out_shape=jax.ShapeDtypeStruct((np_, np_), jnp.float32),
    )(edge_index, edge_index.T)

    S = 64
    while B % S:
        S //= 2
    xflat = jnp.pad(x.reshape(B, n), ((0, 0), (0, np_ - n)),
                    constant_values=_V).reshape(B * np_, 1)

    out = pl.pallas_call(
        _gcn_kernel,
        grid=(B // S,),
        in_specs=[
            pl.BlockSpec((S * np_, 1), lambda i: (i, 0)),
            pl.BlockSpec((np_, np_), lambda i: (0, 0)),
            pl.BlockSpec((_V, H), lambda i: (0, 0)),
            pl.BlockSpec((1, H), lambda i: (0, 0)),
            pl.BlockSpec((L, H, H), lambda i: (0, 0, 0)),
            pl.BlockSpec((L, 1, H), lambda i: (0, 0, 0)),
            pl.BlockSpec((H, 9), lambda i: (0, 0)),
            pl.BlockSpec((1, 9), lambda i: (0, 0)),
        ],
        out_specs=pl.BlockSpec((S * np_, 9), lambda i: (i, 0)),
        out_shape=jax.ShapeDtypeStruct((B * np_, 9), jnp.float32),
    )(xflat, a, W_in, b_in.reshape(1, H), Ws, bs.reshape(L, 1, H),
      W_out, b_out.reshape(1, 9))

    return out.reshape(B, np_, 9)[:, :n, :].reshape(B, 9, 9, 9)


# speedup vs baseline: 5.8290x; 5.8290x over previous
import jax
import jax.numpy as jnp
from jax.experimental import pallas as pl
from jax.experimental.pallas import tpu as pltpu

_N = 81
_NP = 96
_V = 10


def _gcn_kernel(xt_ref, ei_ref, eit_ref, w_in_ref, b_in_ref, ws_ref, bs_ref,
                w_out_ref, b_out_ref, out_ref, a_ref):
    f32 = jnp.float32
    np_ = a_ref.shape[0]
    s = xt_ref.shape[1]

    @pl.when(pl.program_id(0) == 0)
    def _build_adjacency():
        e = ei_ref.shape[1]
        dst_row = ei_ref[1:2, :]
        src_col = eit_ref[:, 0:1]
        dst_col = eit_ref[:, 1:2]
        oh_dt = (jax.lax.broadcasted_iota(jnp.int32, (np_, e), 0)
                 == dst_row).astype(f32)
        lane = jax.lax.broadcasted_iota(jnp.int32, (e, np_), 1)
        oh_s = (src_col == lane).astype(f32)
        oh_d = (dst_col == lane).astype(f32)
        a_u = jnp.dot(oh_dt, oh_s, preferred_element_type=f32)
        ir = jax.lax.broadcasted_iota(jnp.int32, (np_, np_), 0)
        ic = jax.lax.broadcasted_iota(jnp.int32, (np_, np_), 1)
        eye = ((ir == ic) & (ir < _N)).astype(f32)
        a_u = a_u + eye
        deg_row = jnp.dot(jnp.ones((1, e), f32), oh_d,
                          preferred_element_type=f32) + (ic[0:1, :] < _N).astype(f32)
        deg_col = jnp.dot(a_u, jnp.ones((np_, 1), f32),
                          preferred_element_type=f32)
        r_row = jax.lax.rsqrt(jnp.maximum(deg_row, 1.0))
        r_col = jax.lax.rsqrt(jnp.maximum(deg_col, 1.0))
        a_ref[...] = a_u * r_col * r_row

    a = a_ref[...]
    xt = xt_ref[...]
    x_col = jnp.concatenate([xt[:, b:b + 1] for b in range(s)], axis=0)
    oh = (x_col == jax.lax.broadcasted_iota(jnp.int32, (s * np_, _V), 1)
          ).astype(f32)
    h = jnp.maximum(
        jnp.dot(oh, w_in_ref[...], preferred_element_type=f32) + b_in_ref[...],
        0.0)
    for l in range(ws_ref.shape[0]):
        hw = jnp.dot(h, ws_ref[l], preferred_element_type=f32)
        agg = jnp.concatenate(
            [jnp.dot(a, hw[b * np_:(b + 1) * np_, :], preferred_element_type=f32)
             for b in range(s)], axis=0)
        h = jnp.maximum(agg + bs_ref[l], 0.0)
    res = (jnp.dot(h, w_out_ref[...], preferred_element_type=f32)
           + b_out_ref[...])
    for b in range(s):
        base = b * np_
        for i in range(9):
            out_ref[b, i] = res[base + 9 * i:base + 9 * i + 9, :]


def kernel(x, W_in, b_in, Ws, bs, W_out, b_out, edge_index):
    B = x.shape[0]
    H = W_in.shape[1]
    L = Ws.shape[0]
    E = edge_index.shape[1]
    n, np_ = _N, _NP

    S = 128
    while B % S:
        S //= 2
    xt = jnp.pad(x.reshape(B, n), ((0, 0), (0, np_ - n)),
                 constant_values=_V).T

    return pl.pallas_call(
        _gcn_kernel,
        grid=(B // S,),
        in_specs=[
            pl.BlockSpec((np_, S), lambda i: (0, i)),
            pl.BlockSpec((2, E), lambda i: (0, 0)),
            pl.BlockSpec((E, 2), lambda i: (0, 0)),
            pl.BlockSpec((_V, H), lambda i: (0, 0)),
            pl.BlockSpec((1, H), lambda i: (0, 0)),
            pl.BlockSpec((L, H, H), lambda i: (0, 0, 0)),
            pl.BlockSpec((L, 1, H), lambda i: (0, 0, 0)),
            pl.BlockSpec((H, 9), lambda i: (0, 0)),
            pl.BlockSpec((1, 9), lambda i: (0, 0)),
        ],
        out_specs=pl.BlockSpec((S, 9, 9, 9), lambda i: (i, 0, 0, 0)),
        out_shape=jax.ShapeDtypeStruct((B, 9, 9, 9), jnp.float32),
        scratch_shapes=[pltpu.VMEM((np_, np_), jnp.float32)],
    )(xt, edge_index, edge_index.T, W_in, b_in.reshape(1, H), Ws,
      bs.reshape(L, 1, H), W_out, b_out.reshape(1, 9))


# speedup vs baseline: 5.8321x; 1.0005x over previous
import jax
import jax.numpy as jnp
from jax.experimental import pallas as pl
from jax.experimental.pallas import tpu as pltpu

_N = 81
_NP = 96
_V = 10


def _gcn_kernel(xt_ref, ei_ref, eit_ref, w_in_ref, b_in_ref, ws_ref, bs_ref,
                w_out_ref, b_out_ref, out_ref, a_ref):
    f32 = jnp.float32
    np_ = a_ref.shape[0]
    s = xt_ref.shape[1]

    @pl.when(pl.program_id(0) == 0)
    def _build_adjacency():
        e = ei_ref.shape[1]
        dst_row = ei_ref[1:2, :]
        src_col = eit_ref[:, 0:1]
        dst_col = eit_ref[:, 1:2]
        oh_dt = (jax.lax.broadcasted_iota(jnp.int32, (np_, e), 0)
                 == dst_row).astype(f32)
        lane = jax.lax.broadcasted_iota(jnp.int32, (e, np_), 1)
        oh_s = (src_col == lane).astype(f32)
        oh_d = (dst_col == lane).astype(f32)
        a_u = jnp.dot(oh_dt, oh_s, preferred_element_type=f32, precision=jax.lax.Precision.DEFAULT)
        ir = jax.lax.broadcasted_iota(jnp.int32, (np_, np_), 0)
        ic = jax.lax.broadcasted_iota(jnp.int32, (np_, np_), 1)
        eye = ((ir == ic) & (ir < _N)).astype(f32)
        a_u = a_u + eye
        deg_row = jnp.dot(jnp.ones((1, e), f32), oh_d,
                          preferred_element_type=f32, precision=jax.lax.Precision.DEFAULT) + (ic[0:1, :] < _N).astype(f32)
        deg_col = jnp.dot(a_u, jnp.ones((np_, 1), f32),
                          preferred_element_type=f32, precision=jax.lax.Precision.DEFAULT)
        r_row = jax.lax.rsqrt(jnp.maximum(deg_row, 1.0))
        r_col = jax.lax.rsqrt(jnp.maximum(deg_col, 1.0))
        a_ref[...] = a_u * r_col * r_row

    a = a_ref[...]
    xt = xt_ref[...]
    x_col = jnp.concatenate([xt[:, b:b + 1] for b in range(s)], axis=0)
    oh = (x_col == jax.lax.broadcasted_iota(jnp.int32, (s * np_, _V), 1)
          ).astype(f32)
    h = jnp.maximum(
        jnp.dot(oh, w_in_ref[...], preferred_element_type=f32, precision=jax.lax.Precision.DEFAULT) + b_in_ref[...],
        0.0)
    for l in range(ws_ref.shape[0]):
        hw = jnp.dot(h, ws_ref[l], preferred_element_type=f32, precision=jax.lax.Precision.DEFAULT)
        agg = jnp.concatenate(
            [jnp.dot(a, hw[b * np_:(b + 1) * np_, :], preferred_element_type=f32, precision=jax.lax.Precision.DEFAULT)
             for b in range(s)], axis=0)
        h = jnp.maximum(agg + bs_ref[l], 0.0)
    res = (jnp.dot(h, w_out_ref[...], preferred_element_type=f32, precision=jax.lax.Precision.DEFAULT)
           + b_out_ref[...])
    for b in range(s):
        base = b * np_
        for i in range(9):
            out_ref[b, i] = res[base + 9 * i:base + 9 * i + 9, :]


def kernel(x, W_in, b_in, Ws, bs, W_out, b_out, edge_index):
    B = x.shape[0]
    H = W_in.shape[1]
    L = Ws.shape[0]
    E = edge_index.shape[1]
    n, np_ = _N, _NP

    S = 128
    while B % S:
        S //= 2
    xt = jnp.pad(x.reshape(B, n), ((0, 0), (0, np_ - n)),
                 constant_values=_V).T

    return pl.pallas_call(
        _gcn_kernel,
        grid=(B // S,),
        in_specs=[
            pl.BlockSpec((np_, S), lambda i: (0, i)),
            pl.BlockSpec((2, E), lambda i: (0, 0)),
            pl.BlockSpec((E, 2), lambda i: (0, 0)),
            pl.BlockSpec((_V, H), lambda i: (0, 0)),
            pl.BlockSpec((1, H), lambda i: (0, 0)),
            pl.BlockSpec((L, H, H), lambda i: (0, 0, 0)),
            pl.BlockSpec((L, 1, H), lambda i: (0, 0, 0)),
            pl.BlockSpec((H, 9), lambda i: (0, 0)),
            pl.BlockSpec((1, 9), lambda i: (0, 0)),
        ],
        out_specs=pl.BlockSpec((S, 9, 9, 9), lambda i: (i, 0, 0, 0)),
        out_shape=jax.ShapeDtypeStruct((B, 9, 9, 9), jnp.float32),
        scratch_shapes=[pltpu.VMEM((np_, np_), jnp.float32)],
    )(xt, edge_index, edge_index.T, W_in, b_in.reshape(1, H), Ws,
      bs.reshape(L, 1, H), W_out, b_out.reshape(1, 9))


# speedup vs baseline: 5.8439x; 1.0020x over previous
import jax
import jax.numpy as jnp
from jax.experimental import pallas as pl
from jax.experimental.pallas import tpu as pltpu

_N = 81
_NP = 96
_V = 10


def _embed(xt, w_in, b_in):
    f32 = jnp.float32
    np_, s = xt.shape
    x_col = jnp.concatenate([xt[:, b:b + 1] for b in range(s)], axis=0)
    oh = (x_col == jax.lax.broadcasted_iota(jnp.int32, (s * np_, _V), 1)
          ).astype(f32)
    return jnp.maximum(
        jnp.dot(oh, w_in, preferred_element_type=f32) + b_in, 0.0)


def _gcn_kernel(xt_ref, xtn_ref, ei_ref, eit_ref, w_in_ref, b_in_ref, ws_ref,
                bs_ref, w_out_ref, b_out_ref, out_ref, a_ref, h0_ref):
    f32 = jnp.float32
    np_ = a_ref.shape[0]
    s = xt_ref.shape[1]
    pid = pl.program_id(0)

    @pl.when(pid == 0)
    def _build_adjacency():
        e = ei_ref.shape[1]
        dst_row = ei_ref[1:2, :]
        src_col = eit_ref[:, 0:1]
        dst_col = eit_ref[:, 1:2]
        oh_dt = (jax.lax.broadcasted_iota(jnp.int32, (np_, e), 0)
                 == dst_row).astype(f32)
        lane = jax.lax.broadcasted_iota(jnp.int32, (e, np_), 1)
        oh_s = (src_col == lane).astype(f32)
        oh_d = (dst_col == lane).astype(f32)
        a_u = jnp.dot(oh_dt, oh_s, preferred_element_type=f32)
        ir = jax.lax.broadcasted_iota(jnp.int32, (np_, np_), 0)
        ic = jax.lax.broadcasted_iota(jnp.int32, (np_, np_), 1)
        eye = ((ir == ic) & (ir < _N)).astype(f32)
        a_u = a_u + eye
        deg_row = jnp.dot(jnp.ones((1, e), f32), oh_d,
                          preferred_element_type=f32) + (ic[0:1, :] < _N).astype(f32)
        deg_col = jnp.dot(a_u, jnp.ones((np_, 1), f32),
                          preferred_element_type=f32)
        r_row = jax.lax.rsqrt(jnp.maximum(deg_row, 1.0))
        r_col = jax.lax.rsqrt(jnp.maximum(deg_col, 1.0))
        a_ref[...] = a_u * r_col * r_row
        h0_ref[...] = _embed(xt_ref[...], w_in_ref[...], b_in_ref[...])

    a = a_ref[...]
    hw = jnp.dot(h0_ref[...], ws_ref[0], preferred_element_type=f32)

    @pl.when(pid + 1 < pl.num_programs(0))
    def _prefetch_next_embed():
        h0_ref[...] = _embed(xtn_ref[...], w_in_ref[...], b_in_ref[...])

    h = None
    for l in range(ws_ref.shape[0]):
        if l:
            hw = jnp.dot(h, ws_ref[l], preferred_element_type=f32)
        agg = jnp.concatenate(
            [jnp.dot(a, hw[b * np_:(b + 1) * np_, :], preferred_element_type=f32)
             for b in range(s)], axis=0)
        h = jnp.maximum(agg + bs_ref[l], 0.0)
    res = (jnp.dot(h, w_out_ref[...], preferred_element_type=f32)
           + b_out_ref[...])
    for b in range(s):
        base = b * np_
        for i in range(9):
            out_ref[b, i] = res[base + 9 * i:base + 9 * i + 9, :]


def kernel(x, W_in, b_in, Ws, bs, W_out, b_out, edge_index):
    B = x.shape[0]
    H = W_in.shape[1]
    L = Ws.shape[0]
    E = edge_index.shape[1]
    n, np_ = _N, _NP

    S = 128
    while B % S:
        S //= 2
    xt = jnp.pad(x.reshape(B, n), ((0, 0), (0, np_ - n)),
                 constant_values=_V).T

    return pl.pallas_call(
        _gcn_kernel,
        grid=(B // S,),
        in_specs=[
            pl.BlockSpec((np_, S), lambda i: (0, 0)),
            pl.BlockSpec((np_, S),
                         lambda i: (0, jnp.minimum(i + 1, B // S - 1))),
            pl.BlockSpec((2, E), lambda i: (0, 0)),
            pl.BlockSpec((E, 2), lambda i: (0, 0)),
            pl.BlockSpec((_V, H), lambda i: (0, 0)),
            pl.BlockSpec((1, H), lambda i: (0, 0)),
            pl.BlockSpec((L, H, H), lambda i: (0, 0, 0)),
            pl.BlockSpec((L, 1, H), lambda i: (0, 0, 0)),
            pl.BlockSpec((H, 9), lambda i: (0, 0)),
            pl.BlockSpec((1, 9), lambda i: (0, 0)),
        ],
        out_specs=pl.BlockSpec((S, 9, 9, 9), lambda i: (i, 0, 0, 0)),
        out_shape=jax.ShapeDtypeStruct((B, 9, 9, 9), jnp.float32),
        scratch_shapes=[pltpu.VMEM((np_, np_), jnp.float32),
                        pltpu.VMEM((S * np_, H), jnp.float32)],
    )(xt, xt, edge_index, edge_index.T, W_in, b_in.reshape(1, H), Ws,
      bs.reshape(L, 1, H), W_out, b_out.reshape(1, 9))
